# Initial kernel scaffold; baseline (speedup 1.0000x reference)
import jax
import jax.numpy as jnp
from jax.experimental import pallas as pl


def kernel(heatmap):
    raise NotImplementedError("write your pallas kernel here")


# speedup vs baseline: 1.6914x; 1.6914x over previous
import jax
import jax.numpy as jnp
from jax import lax
from jax.experimental import pallas as pl
from jax.experimental.pallas import tpu as pltpu

_K = 100
_BIG = 1 << 30
_CB = 8


def _stage1_kernel(x_ref, s_ref, i_ref):
    x = x_ref[0]
    minf = jnp.float32(-jnp.inf)

    neg_row = jnp.full((_CB, 1, 128), minf, jnp.float32)
    neg_col = jnp.full((_CB, 128, 1), minf, jnp.float32)
    h = jnp.maximum(
        x,
        jnp.maximum(
            jnp.concatenate([x[:, :, 1:], neg_col], axis=2),
            jnp.concatenate([neg_col, x[:, :, :-1]], axis=2),
        ),
    )
    hmax = jnp.maximum(
        h,
        jnp.maximum(
            jnp.concatenate([h[:, 1:, :], neg_row], axis=1),
            jnp.concatenate([neg_row, h[:, :-1, :]], axis=1),
        ),
    )
    hm = jnp.where(hmax == x, x, jnp.float32(0.0))

    lin = (
        lax.broadcasted_iota(jnp.int32, (_CB, 128, 128), 1) * 128
        + lax.broadcasted_iota(jnp.int32, (_CB, 128, 128), 2)
    )
    lane = lax.broadcasted_iota(jnp.int32, (1, 128), 1)

    def body(i, carry):
        xm, sacc, iacc = carry
        m = jnp.max(xm, axis=(1, 2), keepdims=True)
        p = jnp.min(jnp.where(xm == m, lin, _BIG), axis=(1, 2), keepdims=True)
        xm = jnp.where(lin == p, jnp.float32(-1.0), xm)
        sacc = jnp.where(lane == i, m.reshape(_CB, 1), sacc)
        iacc = jnp.where(lane == i, p.reshape(_CB, 1), iacc)
        return xm, sacc, iacc

    _, sacc, iacc = lax.fori_loop(
        0, _K, body,
        (hm, jnp.zeros((_CB, 128), jnp.float32), jnp.zeros((_CB, 128), jnp.int32)),
    )
    s_ref[0] = sacc
    i_ref[0] = iacc


def _stage2_kernel(s_ref, i_ref, oi_ref, oc_ref, oy_ref, ox_ref):
    s = s_ref[0]
    idx = i_ref[0]
    col = lax.broadcasted_iota(jnp.int32, (80, 128), 1)
    row = lax.broadcasted_iota(jnp.int32, (80, 128), 0)
    valid = col < _K
    s = jnp.where(valid, s, jnp.float32(-2.0))
    flat = jnp.where(valid, row * _K + col, _BIG)
    lane = lax.broadcasted_iota(jnp.int32, (1, 128), 1)

    def body(i, carry):
        sm, iacc, cacc = carry
        m = jnp.max(sm)
        p = jnp.min(jnp.where(sm == m, flat, _BIG))
        hit = flat == p
        iv = jnp.sum(jnp.where(hit, idx, 0))
        sm = jnp.where(hit, jnp.float32(-2.0), sm)
        iacc = jnp.where(lane == i, iv, iacc)
        cacc = jnp.where(lane == i, p // _K, cacc)
        return sm, iacc, cacc

    _, iacc, cacc = lax.fori_loop(
        0, _K, body,
        (s, jnp.zeros((1, 128), jnp.int32), jnp.zeros((1, 128), jnp.int32)),
    )
    oi_ref[0] = iacc
    oc_ref[0] = cacc
    oy_ref[0] = (iacc // 128).astype(jnp.float32)
    ox_ref[0] = (iacc % 128).astype(jnp.float32)


def kernel(heatmap):
    BS, C, H, W = heatmap.shape

    s1_scores, s1_idx = pl.pallas_call(
        _stage1_kernel,
        grid=(BS, C // _CB),
        in_specs=[pl.BlockSpec((1, _CB, H, W), lambda b, c: (b, c, 0, 0))],
        out_specs=[
            pl.BlockSpec((1, _CB, 128), lambda b, c: (b, c, 0)),
            pl.BlockSpec((1, _CB, 128), lambda b, c: (b, c, 0)),
        ],
        out_shape=[
            jax.ShapeDtypeStruct((BS, C, 128), jnp.float32),
            jax.ShapeDtypeStruct((BS, C, 128), jnp.int32),
        ],
        compiler_params=pltpu.CompilerParams(
            dimension_semantics=("parallel", "parallel")
        ),
    )(heatmap)

    top_idx, top_cls, top_ys, top_xs = pl.pallas_call(
        _stage2_kernel,
        grid=(BS,),
        in_specs=[
            pl.BlockSpec((1, C, 128), lambda b: (b, 0, 0)),
            pl.BlockSpec((1, C, 128), lambda b: (b, 0, 0)),
        ],
        out_specs=[pl.BlockSpec((1, 1, 128), lambda b: (b, 0, 0))] * 4,
        out_shape=[
            jax.ShapeDtypeStruct((BS, 1, 128), jnp.int32),
            jax.ShapeDtypeStruct((BS, 1, 128), jnp.int32),
            jax.ShapeDtypeStruct((BS, 1, 128), jnp.float32),
            jax.ShapeDtypeStruct((BS, 1, 128), jnp.float32),
        ],
    )(s1_scores, s1_idx)

    return (
        s1_scores[:, :, :_K],
        top_idx[:, 0, :_K],
        top_cls[:, 0, :_K],
        top_ys[:, 0, :_K],
        top_xs[:, 0, :_K],
    )
